# baseline (speedup 1.0000x reference)
import jax
import jax.numpy as jnp
from jax.experimental import pallas as pl
from jax.experimental.pallas import tpu as pltpu

BATCH = 1024
NODE_NUM = 128
INPUT_SZ = 128
HIDDEN_SZ = 128

CB = 128
HALF = BATCH // 2

_PREC_HI = jax.lax.Precision.HIGHEST
_PREC_STEP = jax.lax.Precision.DEFAULT


def _kernel_body(emb0A_ref, emb0B_ref, connTA_ref, connTB_ref, Wcat_ref,
                 bcat_ref, Tf_ref, To_ref, Tz_ref, init_hTA_ref, init_cTA_ref,
                 init_hTB_ref, init_cTB_ref, outA_ref, outB_ref, cA_ref,
                 cB_ref, ghA_ref, gcA_ref, ghB_ref, gcB_ref, semAh, semAc,
                 semBh, semBc):
    H = HIDDEN_SZ
    j = pl.program_id(0)

    cps = [
        pltpu.make_async_copy(init_hTA_ref.at[:, pl.ds(j * CB, CB), :],
                              outA_ref, semAh),
        pltpu.make_async_copy(init_cTA_ref.at[:, pl.ds(j * CB, CB), :],
                              cA_ref, semAc),
        pltpu.make_async_copy(init_hTB_ref.at[:, pl.ds(j * CB, CB), :],
                              outB_ref, semBh),
        pltpu.make_async_copy(init_cTB_ref.at[:, pl.ds(j * CB, CB), :],
                              cB_ref, semBc),
    ]
    for cp in cps:
        cp.start()

    def tt(t_ref):
        t = t_ref[...]
        return jax.lax.dot_general(t, t, (((0,), (0,)), ((), ())),
                                   precision=_PREC_HI,
                                   preferred_element_type=jnp.float32)

    vcat = jnp.concatenate([tt(Tf_ref), tt(To_ref), tt(Tz_ref)], axis=1)

    def proj0(emb_ref):
        foz0 = jax.lax.dot_general(emb_ref[...], Wcat_ref[...],
                                   (((1,), (0,)), ((), ())),
                                   precision=_PREC_HI,
                                   preferred_element_type=jnp.float32)
        return foz0 + bcat_ref[...]

    def node0(foz0):
        f = jax.nn.sigmoid(foz0[:, 0:H])
        o = jax.nn.sigmoid(foz0[:, H:2 * H])
        z = jnp.tanh(foz0[:, 2 * H:3 * H])
        c0 = z * (1.0 - f)
        h0 = o * jnp.tanh(c0)
        return h0, c0

    foz0A = proj0(emb0A_ref)
    foz0B = proj0(emb0B_ref)
    h0A, c0A = node0(foz0A)
    h0B, c0B = node0(foz0B)

    for cp in cps:
        cp.wait()
    outA_ref[pl.ds(0, 1), :, :] = h0A[None]
    cA_ref[pl.ds(0, 1), :, :] = c0A[None]
    outB_ref[pl.ds(0, 1), :, :] = h0B[None]
    cB_ref[pl.ds(0, 1), :, :] = c0B[None]

    def gather(i, connT_ref, out_ref, c_ref, gh_ref, gc_ref):
        for b in range(CB):
            p = connT_ref[i, b]
            gh_ref[:, pl.ds(b, 1), :] = out_ref[pl.ds(p, 1), pl.ds(b, 1), :]
            gc_ref[:, pl.ds(b, 1), :] = c_ref[pl.ds(p, 1), pl.ds(b, 1), :]

    def cell(gh_ref, gc_ref, vcat, foz0):
        gates = jax.lax.dot_general(gh_ref[0], vcat, (((1,), (0,)), ((), ())),
                                    precision=_PREC_STEP,
                                    preferred_element_type=jnp.float32)
        gates = gates + foz0
        f = jax.nn.sigmoid(gates[:, 0:H])
        o = jax.nn.sigmoid(gates[:, H:2 * H])
        z = jnp.tanh(gates[:, 2 * H:3 * H])
        c = gc_ref[0] * f + z * (1.0 - f)
        h = o * jnp.tanh(c)
        return h, c

    def step(i, _):
        gather(i, connTA_ref, outA_ref, cA_ref, ghA_ref, gcA_ref)
        gather(i, connTB_ref, outB_ref, cB_ref, ghB_ref, gcB_ref)
        hA, cA = cell(ghA_ref, gcA_ref, vcat, foz0A)
        hB, cB = cell(ghB_ref, gcB_ref, vcat, foz0B)
        outA_ref[pl.ds(i, 1), :, :] = hA[None]
        cA_ref[pl.ds(i, 1), :, :] = cA[None]
        outB_ref[pl.ds(i, 1), :, :] = hB[None]
        cB_ref[pl.ds(i, 1), :, :] = cB[None]
        return 0

    jax.lax.fori_loop(1, NODE_NUM, step, 0)


def kernel(tree_embedding, node_connection, node_mask, W_f, b_f, W_o, b_o,
           W_z, b_z, T_f, T_o, T_z, init_h, init_c):
    del node_mask
    H = HIDDEN_SZ
    emb0 = tree_embedding[:, 0, :]
    connT = jnp.swapaxes(node_connection, 0, 1)
    Wcat = jnp.concatenate([W_f.T, W_o.T, W_z.T], axis=1)
    bcat = jnp.concatenate([b_f, b_o, b_z]).reshape(1, 3 * H)
    init_hTA = jnp.swapaxes(init_h[:HALF], 0, 1)
    init_cTA = jnp.swapaxes(init_c[:HALF], 0, 1)
    init_hTB = jnp.swapaxes(init_h[HALF:], 0, 1)
    init_cTB = jnp.swapaxes(init_c[HALF:], 0, 1)

    grid = (HALF // CB,)
    vs = pl.BlockSpec((NODE_NUM, CB, H), lambda j: (0, j, 0))
    wspec = [
        pl.BlockSpec((INPUT_SZ, 3 * H), lambda j: (0, 0)),
        pl.BlockSpec((1, 3 * H), lambda j: (0, 0)),
        pl.BlockSpec((H, H), lambda j: (0, 0)),
        pl.BlockSpec((H, H), lambda j: (0, 0)),
        pl.BlockSpec((H, H), lambda j: (0, 0)),
    ]
    outA, outB = pl.pallas_call(
        _kernel_body,
        grid=grid,
        in_specs=[
            pl.BlockSpec((CB, INPUT_SZ), lambda j: (j, 0)),
            pl.BlockSpec((CB, INPUT_SZ), lambda j: (j, 0)),
            pl.BlockSpec((NODE_NUM, CB), lambda j: (0, j),
                         memory_space=pltpu.SMEM),
            pl.BlockSpec((NODE_NUM, CB), lambda j: (0, j),
                         memory_space=pltpu.SMEM),
        ] + wspec + [
            pl.BlockSpec(memory_space=pltpu.MemorySpace.HBM),
            pl.BlockSpec(memory_space=pltpu.MemorySpace.HBM),
            pl.BlockSpec(memory_space=pltpu.MemorySpace.HBM),
            pl.BlockSpec(memory_space=pltpu.MemorySpace.HBM),
        ],
        out_specs=[vs, vs],
        out_shape=[
            jax.ShapeDtypeStruct((NODE_NUM, HALF, H), jnp.float32),
            jax.ShapeDtypeStruct((NODE_NUM, HALF, H), jnp.float32),
        ],
        scratch_shapes=[
            pltpu.VMEM((NODE_NUM, CB, H), jnp.float32),
            pltpu.VMEM((NODE_NUM, CB, H), jnp.float32),
            pltpu.VMEM((1, CB, H), jnp.float32),
            pltpu.VMEM((1, CB, H), jnp.float32),
            pltpu.VMEM((1, CB, H), jnp.float32),
            pltpu.VMEM((1, CB, H), jnp.float32),
            pltpu.SemaphoreType.DMA,
            pltpu.SemaphoreType.DMA,
            pltpu.SemaphoreType.DMA,
            pltpu.SemaphoreType.DMA,
        ],
    )(emb0[:HALF], emb0[HALF:], connT[:, :HALF], connT[:, HALF:], Wcat, bcat,
      T_f, T_o, T_z, init_hTA, init_cTA, init_hTB, init_cTB)
    return jnp.concatenate(
        [jnp.swapaxes(outA, 0, 1), jnp.swapaxes(outB, 0, 1)], axis=0)


# speedup vs baseline: 1.1404x; 1.1404x over previous
import jax
import jax.numpy as jnp
from jax.experimental import pallas as pl
from jax.experimental.pallas import tpu as pltpu

BATCH = 1024
NODE_NUM = 128
INPUT_SZ = 128
HIDDEN_SZ = 128

BB = 256

_PREC_HI = jax.lax.Precision.HIGHEST
_PREC_STEP = jax.lax.Precision.DEFAULT


def _kernel_body(emb0_ref, connT_ref, Wcat_ref, bcat_ref, Tf_ref, To_ref,
                 Tz_ref, init_hT_ref, init_cT_ref, out_ref, c_ref, gh_ref,
                 gc_ref, sem_h, sem_c):
    H = HIDDEN_SZ
    j = pl.program_id(0)

    cp_h = pltpu.make_async_copy(init_hT_ref.at[:, pl.ds(j * BB, BB), :],
                                 out_ref, sem_h)
    cp_c = pltpu.make_async_copy(init_cT_ref.at[:, pl.ds(j * BB, BB), :],
                                 c_ref, sem_c)
    cp_h.start()
    cp_c.start()

    def tt(t_ref):
        t = t_ref[...]
        return jax.lax.dot_general(t, t, (((0,), (0,)), ((), ())),
                                   precision=_PREC_HI,
                                   preferred_element_type=jnp.float32)

    vcat = jnp.concatenate([tt(Tf_ref), tt(To_ref), tt(Tz_ref)], axis=1)

    foz0 = jax.lax.dot_general(emb0_ref[...], Wcat_ref[...],
                               (((1,), (0,)), ((), ())), precision=_PREC_HI,
                               preferred_element_type=jnp.float32)
    foz0 = foz0 + bcat_ref[...]
    f0 = foz0[:, 0:H]
    o0 = foz0[:, H:2 * H]
    z0 = foz0[:, 2 * H:3 * H]

    f = jax.nn.sigmoid(f0)
    o = jax.nn.sigmoid(o0)
    z = jnp.tanh(z0)
    c0 = z * (1.0 - f)
    h0 = o * jnp.tanh(c0)

    cp_h.wait()
    cp_c.wait()
    out_ref[pl.ds(0, 1), :, :] = h0[None]
    c_ref[pl.ds(0, 1), :, :] = c0[None]

    def step(i, _):
        for b in range(BB):
            p = connT_ref[i, b]
            gh_ref[:, pl.ds(b, 1), :] = out_ref[pl.ds(p, 1), pl.ds(b, 1), :]
            gc_ref[:, pl.ds(b, 1), :] = c_ref[pl.ds(p, 1), pl.ds(b, 1), :]
        gates = jax.lax.dot_general(gh_ref[0], vcat, (((1,), (0,)), ((), ())),
                                    precision=_PREC_STEP,
                                    preferred_element_type=jnp.float32)
        gates = gates + foz0
        f = jax.nn.sigmoid(gates[:, 0:H])
        o = jax.nn.sigmoid(gates[:, H:2 * H])
        z = jnp.tanh(gates[:, 2 * H:3 * H])
        c = gc_ref[0] * f + z * (1.0 - f)
        h = o * jnp.tanh(c)
        out_ref[pl.ds(i, 1), :, :] = h[None]
        c_ref[pl.ds(i, 1), :, :] = c[None]
        return 0

    jax.lax.fori_loop(1, NODE_NUM, step, 0, unroll=2)


def kernel(tree_embedding, node_connection, node_mask, W_f, b_f, W_o, b_o,
           W_z, b_z, T_f, T_o, T_z, init_h, init_c):
    del node_mask
    H = HIDDEN_SZ
    emb0 = tree_embedding[:, 0, :]
    connT = jnp.swapaxes(node_connection, 0, 1)
    Wcat = jnp.concatenate([W_f.T, W_o.T, W_z.T], axis=1)
    bcat = jnp.concatenate([b_f, b_o, b_z]).reshape(1, 3 * H)
    init_hT = jnp.swapaxes(init_h, 0, 1)
    init_cT = jnp.swapaxes(init_c, 0, 1)

    nb = BATCH // BB
    grid = (nb,)
    out = pl.pallas_call(
        _kernel_body,
        grid=grid,
        in_specs=[
            pl.BlockSpec((BB, INPUT_SZ), lambda j: (j, 0)),
            pl.BlockSpec((NODE_NUM, BB), lambda j: (0, j),
                         memory_space=pltpu.SMEM),
            pl.BlockSpec((INPUT_SZ, 3 * H), lambda j: (0, 0)),
            pl.BlockSpec((1, 3 * H), lambda j: (0, 0)),
            pl.BlockSpec((H, H), lambda j: (0, 0)),
            pl.BlockSpec((H, H), lambda j: (0, 0)),
            pl.BlockSpec((H, H), lambda j: (0, 0)),
            pl.BlockSpec(memory_space=pltpu.MemorySpace.HBM),
            pl.BlockSpec(memory_space=pltpu.MemorySpace.HBM),
        ],
        out_specs=pl.BlockSpec((NODE_NUM, BB, H), lambda j: (0, j, 0)),
        out_shape=jax.ShapeDtypeStruct((NODE_NUM, BATCH, H), jnp.float32),
        scratch_shapes=[
            pltpu.VMEM((NODE_NUM, BB, H), jnp.float32),
            pltpu.VMEM((1, BB, H), jnp.float32),
            pltpu.VMEM((1, BB, H), jnp.float32),
            pltpu.SemaphoreType.DMA,
            pltpu.SemaphoreType.DMA,
        ],
    )(emb0, connT, Wcat, bcat, T_f, T_o, T_z, init_hT, init_cT)
    return jnp.swapaxes(out, 0, 1)


# speedup vs baseline: 1.1977x; 1.0502x over previous
import jax
import jax.numpy as jnp
from jax.experimental import pallas as pl
from jax.experimental.pallas import tpu as pltpu

BATCH = 1024
NODE_NUM = 128
INPUT_SZ = 128
HIDDEN_SZ = 128

BB = 256

_PREC_HI = jax.lax.Precision.HIGHEST
_PREC_STEP = jax.lax.Precision.DEFAULT


def _kernel_body(emb0_ref, connT_ref, Wcat_ref, bcat_ref, Tf_ref, To_ref,
                 Tz_ref, init_hT_ref, init_cT_ref, out_ref, c_ref, gh_ref,
                 gc_ref, sem_h, sem_c):
    H = HIDDEN_SZ
    j = pl.program_id(0)

    cp_h = pltpu.make_async_copy(init_hT_ref.at[:, pl.ds(j * BB, BB), :],
                                 out_ref, sem_h)
    cp_c = pltpu.make_async_copy(init_cT_ref.at[:, pl.ds(j * BB, BB), :],
                                 c_ref, sem_c)
    cp_h.start()
    cp_c.start()

    def tt(t_ref):
        t = t_ref[...]
        return jax.lax.dot_general(t, t, (((0,), (0,)), ((), ())),
                                   precision=_PREC_HI,
                                   preferred_element_type=jnp.float32)

    vcat = jnp.concatenate([tt(Tf_ref), tt(To_ref), tt(Tz_ref)], axis=1)

    foz0 = jax.lax.dot_general(emb0_ref[...], Wcat_ref[...],
                               (((1,), (0,)), ((), ())), precision=_PREC_HI,
                               preferred_element_type=jnp.float32)
    foz0 = foz0 + bcat_ref[...]
    f0 = foz0[:, 0:H]
    o0 = foz0[:, H:2 * H]
    z0 = foz0[:, 2 * H:3 * H]

    f = jax.nn.sigmoid(f0)
    o = jax.nn.sigmoid(o0)
    z = jnp.tanh(z0)
    c0 = z * (1.0 - f)
    h0 = o * jnp.tanh(c0)

    cp_h.wait()
    cp_c.wait()
    out_ref[pl.ds(0, 1), :, :] = h0[None]
    c_ref[pl.ds(0, 1), :, :] = c0[None]

    def step(i, _):
        for b in range(BB):
            p = connT_ref[i, b]
            gh_ref[:, pl.ds(b, 1), :] = out_ref[pl.ds(p, 1), pl.ds(b, 1), :]
            gc_ref[:, pl.ds(b, 1), :] = c_ref[pl.ds(p, 1), pl.ds(b, 1), :]
        gates = jax.lax.dot_general(gh_ref[0], vcat, (((1,), (0,)), ((), ())),
                                    precision=_PREC_STEP,
                                    preferred_element_type=jnp.float32)
        gates = gates + foz0
        f = jax.nn.sigmoid(gates[:, 0:H])
        o = jax.nn.sigmoid(gates[:, H:2 * H])
        z = jnp.tanh(gates[:, 2 * H:3 * H])
        c = gc_ref[0] * f + z * (1.0 - f)
        h = o * jnp.tanh(c)
        out_ref[pl.ds(i, 1), :, :] = h[None]
        c_ref[pl.ds(i, 1), :, :] = c[None]
        return 0

    jax.lax.fori_loop(1, NODE_NUM, step, 0)


def kernel(tree_embedding, node_connection, node_mask, W_f, b_f, W_o, b_o,
           W_z, b_z, T_f, T_o, T_z, init_h, init_c):
    del node_mask
    H = HIDDEN_SZ
    emb0 = tree_embedding[:, 0, :]
    connT = jnp.swapaxes(node_connection, 0, 1)
    Wcat = jnp.concatenate([W_f.T, W_o.T, W_z.T], axis=1)
    bcat = jnp.concatenate([b_f, b_o, b_z]).reshape(1, 3 * H)
    init_hT = jnp.swapaxes(init_h, 0, 1)
    init_cT = jnp.swapaxes(init_c, 0, 1)

    nb = BATCH // BB
    grid = (nb,)
    out = pl.pallas_call(
        _kernel_body,
        grid=grid,
        in_specs=[
            pl.BlockSpec((BB, INPUT_SZ), lambda j: (j, 0)),
            pl.BlockSpec((NODE_NUM, BB), lambda j: (0, j),
                         memory_space=pltpu.SMEM),
            pl.BlockSpec((INPUT_SZ, 3 * H), lambda j: (0, 0)),
            pl.BlockSpec((1, 3 * H), lambda j: (0, 0)),
            pl.BlockSpec((H, H), lambda j: (0, 0)),
            pl.BlockSpec((H, H), lambda j: (0, 0)),
            pl.BlockSpec((H, H), lambda j: (0, 0)),
            pl.BlockSpec(memory_space=pltpu.MemorySpace.HBM),
            pl.BlockSpec(memory_space=pltpu.MemorySpace.HBM),
        ],
        out_specs=pl.BlockSpec((NODE_NUM, BB, H), lambda j: (0, j, 0)),
        out_shape=jax.ShapeDtypeStruct((NODE_NUM, BATCH, H), jnp.float32),
        scratch_shapes=[
            pltpu.VMEM((NODE_NUM, BB, H), jnp.float32),
            pltpu.VMEM((1, BB, H), jnp.float32),
            pltpu.VMEM((1, BB, H), jnp.float32),
            pltpu.SemaphoreType.DMA,
            pltpu.SemaphoreType.DMA,
        ],
    )(emb0, connT, Wcat, bcat, T_f, T_o, T_z, init_hT, init_cT)
    return jnp.swapaxes(out, 0, 1)


# speedup vs baseline: 1.2988x; 1.0844x over previous
import jax
import jax.numpy as jnp
from jax.experimental import pallas as pl
from jax.experimental.pallas import tpu as pltpu

BATCH = 1024
NODE_NUM = 128
INPUT_SZ = 128
HIDDEN_SZ = 128

BB = 256
NCH = 8

_PREC_HI = jax.lax.Precision.HIGHEST
_PREC_STEP = jax.lax.Precision.DEFAULT


def _kernel_body(emb0_ref, connT_ref, Wcat_ref, bcat_ref, Tf_ref, To_ref,
                 Tz_ref, init_h_ref, init_c_ref, out_ref, c_ref, gh_ref,
                 gc_ref, stage_ref, sem0, sem1):
    H = HIDDEN_SZ
    j = pl.program_id(0)
    nk = NODE_NUM // NCH
    sems = [sem0, sem1]

    def load_init(src_ref, dst_ref):
        def mk(k):
            return pltpu.make_async_copy(
                src_ref.at[pl.ds(j * BB, BB), pl.ds(k * NCH, NCH), :],
                stage_ref.at[k % 2], sems[k % 2])

        mk(0).start()
        mk(1).start()
        for k in range(nk):
            mk(k).wait()
            dst_ref[pl.ds(k * NCH, NCH), :, :] = jnp.transpose(
                stage_ref[k % 2], (1, 0, 2))
            if k + 2 < nk:
                mk(k + 2).start()

    def tt(t_ref):
        t = t_ref[...]
        return jax.lax.dot_general(t, t, (((0,), (0,)), ((), ())),
                                   precision=_PREC_HI,
                                   preferred_element_type=jnp.float32)

    vcat = jnp.concatenate([tt(Tf_ref), tt(To_ref), tt(Tz_ref)], axis=1)

    foz0 = jax.lax.dot_general(emb0_ref[...], Wcat_ref[...],
                               (((1,), (0,)), ((), ())), precision=_PREC_HI,
                               preferred_element_type=jnp.float32)
    foz0 = foz0 + bcat_ref[...]
    f0 = foz0[:, 0:H]
    o0 = foz0[:, H:2 * H]
    z0 = foz0[:, 2 * H:3 * H]

    f = jax.nn.sigmoid(f0)
    o = jax.nn.sigmoid(o0)
    z = jnp.tanh(z0)
    c0 = z * (1.0 - f)
    h0 = o * jnp.tanh(c0)

    load_init(init_h_ref, out_ref)
    load_init(init_c_ref, c_ref)
    out_ref[pl.ds(0, 1), :, :] = h0[None]
    c_ref[pl.ds(0, 1), :, :] = c0[None]

    def step(i, _):
        for b in range(BB):
            p = connT_ref[i, b]
            gh_ref[:, pl.ds(b, 1), :] = out_ref[pl.ds(p, 1), pl.ds(b, 1), :]
            gc_ref[:, pl.ds(b, 1), :] = c_ref[pl.ds(p, 1), pl.ds(b, 1), :]
        gates = jax.lax.dot_general(gh_ref[0], vcat, (((1,), (0,)), ((), ())),
                                    precision=_PREC_STEP,
                                    preferred_element_type=jnp.float32)
        gates = gates + foz0
        f = jax.nn.sigmoid(gates[:, 0:H])
        o = jax.nn.sigmoid(gates[:, H:2 * H])
        z = jnp.tanh(gates[:, 2 * H:3 * H])
        c = gc_ref[0] * f + z * (1.0 - f)
        h = o * jnp.tanh(c)
        out_ref[pl.ds(i, 1), :, :] = h[None]
        c_ref[pl.ds(i, 1), :, :] = c[None]
        return 0

    jax.lax.fori_loop(1, NODE_NUM, step, 0)


def kernel(tree_embedding, node_connection, node_mask, W_f, b_f, W_o, b_o,
           W_z, b_z, T_f, T_o, T_z, init_h, init_c):
    del node_mask
    H = HIDDEN_SZ
    emb0 = tree_embedding[:, 0, :]
    connT = jnp.swapaxes(node_connection, 0, 1)
    Wcat = jnp.concatenate([W_f.T, W_o.T, W_z.T], axis=1)
    bcat = jnp.concatenate([b_f, b_o, b_z]).reshape(1, 3 * H)

    nb = BATCH // BB
    grid = (nb,)
    out = pl.pallas_call(
        _kernel_body,
        grid=grid,
        in_specs=[
            pl.BlockSpec((BB, INPUT_SZ), lambda j: (j, 0)),
            pl.BlockSpec((NODE_NUM, BB), lambda j: (0, j),
                         memory_space=pltpu.SMEM),
            pl.BlockSpec((INPUT_SZ, 3 * H), lambda j: (0, 0)),
            pl.BlockSpec((1, 3 * H), lambda j: (0, 0)),
            pl.BlockSpec((H, H), lambda j: (0, 0)),
            pl.BlockSpec((H, H), lambda j: (0, 0)),
            pl.BlockSpec((H, H), lambda j: (0, 0)),
            pl.BlockSpec(memory_space=pltpu.MemorySpace.HBM),
            pl.BlockSpec(memory_space=pltpu.MemorySpace.HBM),
        ],
        out_specs=pl.BlockSpec((NODE_NUM, BB, H), lambda j: (0, j, 0)),
        out_shape=jax.ShapeDtypeStruct((NODE_NUM, BATCH, H), jnp.float32),
        scratch_shapes=[
            pltpu.VMEM((NODE_NUM, BB, H), jnp.float32),
            pltpu.VMEM((1, BB, H), jnp.float32),
            pltpu.VMEM((1, BB, H), jnp.float32),
            pltpu.VMEM((2, BB, NCH, H), jnp.float32),
            pltpu.SemaphoreType.DMA,
            pltpu.SemaphoreType.DMA,
        ],
    )(emb0, connT, Wcat, bcat, T_f, T_o, T_z, init_h, init_c)
    return jnp.swapaxes(out, 0, 1)


# speedup vs baseline: 1.4424x; 1.1106x over previous
import jax
import jax.numpy as jnp
from jax.experimental import pallas as pl
from jax.experimental.pallas import tpu as pltpu

BATCH = 1024
NODE_NUM = 128
INPUT_SZ = 128
HIDDEN_SZ = 128

BB = 256
NCH = 8

_PREC_HI = jax.lax.Precision.HIGHEST
_PREC_STEP = jax.lax.Precision.DEFAULT


def _kernel_body(emb0_ref, connT_ref, Wcat_ref, bcat_ref, Tf_ref, To_ref,
                 Tz_ref, init_h_ref, init_c_ref, out_ref, sh_ref, c_ref,
                 gh_ref, gc_ref, stage_ref, sem0, sem1):
    H = HIDDEN_SZ
    j = pl.program_id(0)
    nk = NODE_NUM // NCH
    sems = [sem0, sem1]

    def load_init(src_ref, dst_ref):
        def mk(k):
            return pltpu.make_async_copy(
                src_ref.at[pl.ds(j * BB, BB), pl.ds(k * NCH, NCH), :],
                stage_ref.at[k % 2], sems[k % 2])

        mk(0).start()
        mk(1).start()
        for k in range(nk):
            mk(k).wait()
            dst_ref[pl.ds(k * NCH, NCH), :, :] = jnp.transpose(
                stage_ref[k % 2], (1, 0, 2))
            if k + 2 < nk:
                mk(k + 2).start()

    def tt(t_ref):
        t = t_ref[...]
        return jax.lax.dot_general(t, t, (((0,), (0,)), ((), ())),
                                   precision=_PREC_HI,
                                   preferred_element_type=jnp.float32)

    vcat = jnp.concatenate([tt(Tf_ref), tt(To_ref), tt(Tz_ref)], axis=1)

    foz0 = jax.lax.dot_general(emb0_ref[...], Wcat_ref[...],
                               (((1,), (0,)), ((), ())), precision=_PREC_HI,
                               preferred_element_type=jnp.float32)
    foz0 = foz0 + bcat_ref[...]
    f0 = foz0[:, 0:H]
    o0 = foz0[:, H:2 * H]
    z0 = foz0[:, 2 * H:3 * H]

    f = jax.nn.sigmoid(f0)
    o = jax.nn.sigmoid(o0)
    z = jnp.tanh(z0)
    c0 = z * (1.0 - f)
    h0 = o * jnp.tanh(c0)

    load_init(init_h_ref, sh_ref)
    load_init(init_c_ref, c_ref)
    sh_ref[pl.ds(0, 1), :, :] = h0[None]
    c_ref[pl.ds(0, 1), :, :] = c0[None]

    def step(i, _):
        for b in range(BB):
            p = connT_ref[i, b]
            gh_ref[:, pl.ds(b, 1), :] = sh_ref[pl.ds(p, 1), pl.ds(b, 1), :]
            gc_ref[:, pl.ds(b, 1), :] = c_ref[pl.ds(p, 1), pl.ds(b, 1), :]
        gates = jax.lax.dot_general(gh_ref[0], vcat, (((1,), (0,)), ((), ())),
                                    precision=_PREC_STEP,
                                    preferred_element_type=jnp.float32)
        gates = gates + foz0
        f = jax.nn.sigmoid(gates[:, 0:H])
        o = jax.nn.sigmoid(gates[:, H:2 * H])
        z = jnp.tanh(gates[:, 2 * H:3 * H])
        c = gc_ref[0] * f + z * (1.0 - f)
        h = o * jnp.tanh(c)
        sh_ref[pl.ds(i, 1), :, :] = h[None]
        c_ref[pl.ds(i, 1), :, :] = c[None]
        return 0

    jax.lax.fori_loop(1, NODE_NUM, step, 0)

    def mko(k):
        return pltpu.make_async_copy(
            stage_ref.at[k % 2],
            out_ref.at[pl.ds(j * BB, BB), pl.ds(k * NCH, NCH), :],
            sems[k % 2])

    for k in range(nk):
        if k >= 2:
            mko(k - 2).wait()
        stage_ref[k % 2] = jnp.transpose(sh_ref[pl.ds(k * NCH, NCH), :, :],
                                         (1, 0, 2))
        mko(k).start()
    mko(nk - 2).wait()
    mko(nk - 1).wait()


def kernel(tree_embedding, node_connection, node_mask, W_f, b_f, W_o, b_o,
           W_z, b_z, T_f, T_o, T_z, init_h, init_c):
    del node_mask
    H = HIDDEN_SZ
    emb0 = tree_embedding[:, 0, :]
    connT = jnp.swapaxes(node_connection, 0, 1)
    Wcat = jnp.concatenate([W_f.T, W_o.T, W_z.T], axis=1)
    bcat = jnp.concatenate([b_f, b_o, b_z]).reshape(1, 3 * H)

    nb = BATCH // BB
    grid = (nb,)
    out = pl.pallas_call(
        _kernel_body,
        grid=grid,
        in_specs=[
            pl.BlockSpec((BB, INPUT_SZ), lambda j: (j, 0)),
            pl.BlockSpec((NODE_NUM, BB), lambda j: (0, j),
                         memory_space=pltpu.SMEM),
            pl.BlockSpec((INPUT_SZ, 3 * H), lambda j: (0, 0)),
            pl.BlockSpec((1, 3 * H), lambda j: (0, 0)),
            pl.BlockSpec((H, H), lambda j: (0, 0)),
            pl.BlockSpec((H, H), lambda j: (0, 0)),
            pl.BlockSpec((H, H), lambda j: (0, 0)),
            pl.BlockSpec(memory_space=pltpu.MemorySpace.HBM),
            pl.BlockSpec(memory_space=pltpu.MemorySpace.HBM),
        ],
        out_specs=pl.BlockSpec(memory_space=pltpu.MemorySpace.HBM),
        out_shape=jax.ShapeDtypeStruct((BATCH, NODE_NUM, H), jnp.float32),
        scratch_shapes=[
            pltpu.VMEM((NODE_NUM, BB, H), jnp.float32),
            pltpu.VMEM((NODE_NUM, BB, H), jnp.float32),
            pltpu.VMEM((1, BB, H), jnp.float32),
            pltpu.VMEM((1, BB, H), jnp.float32),
            pltpu.VMEM((2, BB, NCH, H), jnp.float32),
            pltpu.SemaphoreType.DMA,
            pltpu.SemaphoreType.DMA,
        ],
    )(emb0, connT, Wcat, bcat, T_f, T_o, T_z, init_h, init_c)
    return out
